# baseline (speedup 1.0000x reference)
import functools

import jax
import jax.numpy as jnp
from jax import lax
from jax.experimental import pallas as pl
from jax.experimental.pallas import tpu as pltpu
from jax.experimental.pallas import tpu_sc as plsc

N = 10000
E = 320000
D = 128
DE = 16
NCHIR = 4

NC = 2
NS = 16
NW = NC * NS
EPW = E // NW
BE = 40
NBLK = EPW // BE
RPT = 640
NPW = 312

@functools.cache
def _embed_sc_build():
    mesh = plsc.VectorSubcoreMesh(core_axis_name="c", subcore_axis_name="s")
    return functools.partial(
        pl.kernel,
        mesh=mesh,
        out_type=jax.ShapeDtypeStruct((N, D), jnp.float32),
        scratch_types=[
            pltpu.VMEM((NPW,), jnp.int32),
            pltpu.VMEM((NPW, D), jnp.float32),
            pltpu.VMEM((16,), jnp.int32),
            pltpu.VMEM((16, D), jnp.float32),
            pltpu.SemaphoreType.DMA,
        ],
    )(_embed_sc_body)


def _embed_sc_body(tbl_hbm, key_hbm, x_hbm, idx_v, rows_v, idx2_v, rows2_v,
                   sem):
    wid = lax.axis_index("s") * NC + lax.axis_index("c")
    base = wid * NPW
    pltpu.sync_copy(key_hbm.at[pl.ds(base, NPW)], idx_v)
    for c in range(3):
        pltpu.async_copy(tbl_hbm.at[idx_v.at[pl.ds(c * 104, 104)]],
                         rows_v.at[pl.ds(c * 104, 104)], sem).wait()
    pltpu.sync_copy(rows_v, x_hbm.at[pl.ds(base, NPW)])

    @pl.when(wid == NW - 1)
    def _():
        pltpu.sync_copy(key_hbm.at[pl.ds(NW * NPW, 16)], idx2_v)
        pltpu.async_copy(tbl_hbm.at[idx2_v], rows2_v, sem).wait()
        pltpu.sync_copy(rows2_v, x_hbm.at[pl.ds(NW * NPW, 16)])


@functools.cache
def _msg_aggr_sc_build():
    mesh = plsc.VectorSubcoreMesh(core_axis_name="c", subcore_axis_name="s")
    return functools.partial(
        pl.kernel,
        mesh=mesh,
        out_type=jax.ShapeDtypeStruct((NC, N, D), jnp.float32),
        scratch_types=[
            pltpu.VMEM((4, 2, BE), jnp.int32),
            pltpu.VMEM((3, BE, D), jnp.float32),
            pltpu.VMEM((3, BE, D), jnp.float32),
            pltpu.VMEM_SHARED((N, D), jnp.float32),
            pltpu.SemaphoreType.DMA((4,)),
            pltpu.SemaphoreType.DMA((3,)),
            pltpu.SemaphoreType.DMA((3,)),
            pltpu.SemaphoreType.DMA((3,)),
        ],
    )(_msg_aggr_sc_body)


def _msg_aggr_sc_body(h_hbm, e_hbm, ei_hbm, out_hbm,
                      ibuf, hbuf, ebuf, acc, sem_i, sem_g, sem_e, sem_s):
    cid = lax.axis_index("c")
    sid = lax.axis_index("s")
    wid = sid * NC + cid
    nrows = jnp.where(sid == NS - 1, 400, RPT)
    nch = nrows // BE
    zv = jnp.zeros((16,), jnp.float32)

    def zrow(r, _):
        for g in range(8):
            hbuf[0, r, pl.ds(g * 16, 16)] = zv
        return 0

    lax.fori_loop(0, BE, zrow, 0)

    def zch(c, _):
        pltpu.sync_copy(hbuf.at[0], acc.at[pl.ds(sid * RPT + c * BE, BE)])
        return 0

    lax.fori_loop(0, nch, zch, 0)

    def issue_idx(j):
        pltpu.async_copy(ei_hbm.at[wid, j], ibuf.at[j % 4], sem_i.at[j % 4])

    def issue_body(j):
        s = j % 4
        b = j % 3
        pltpu.async_copy(h_hbm.at[ibuf.at[s, 0]], hbuf.at[b], sem_g.at[b])
        pltpu.async_copy(e_hbm.at[pl.ds(wid * EPW + j * BE, BE)],
                         ebuf.at[b], sem_e.at[b])

    issue_idx(0)
    pltpu.make_async_copy(ei_hbm.at[wid, 0], ibuf.at[0], sem_i.at[0]).wait()
    issue_body(0)
    issue_idx(1)
    pltpu.make_async_copy(ei_hbm.at[wid, 1], ibuf.at[1], sem_i.at[1]).wait()
    issue_body(1)
    issue_idx(2)
    plsc.subcore_barrier()

    def block(j, _):
        s = j % 4
        b = j % 3
        pltpu.make_async_copy(h_hbm.at[ibuf.at[s, 0]], hbuf.at[b],
                              sem_g.at[b]).wait()
        pltpu.make_async_copy(e_hbm.at[pl.ds(wid * EPW + j * BE, BE)],
                              ebuf.at[b], sem_e.at[b]).wait()

        for bb in range(3):
            @pl.when(b == bb)
            def _(bb=bb):
                def mrow(r, _):
                    sls = [pl.ds(g * 16, 16) for g in range(8)]
                    hv = [hbuf[bb, r, sl] for sl in sls]
                    ev = [ebuf[bb, r, sl] for sl in sls]
                    mv = [jnp.maximum(h + e, 0.0) for h, e in zip(hv, ev)]
                    for sl, m in zip(sls, mv):
                        hbuf[bb, r, sl] = m
                    return 0

                lax.fori_loop(0, BE, mrow, 0)
        pltpu.async_copy(hbuf.at[b], acc.at[ibuf.at[s, 1]], sem_s.at[b],
                         add=True)

        @pl.when(j + 2 < NBLK)
        def _():
            ns = (j + 2) % 4
            nb = (j + 2) % 3
            pltpu.make_async_copy(ei_hbm.at[wid, j + 2], ibuf.at[ns],
                                  sem_i.at[ns]).wait()
            @pl.when(j >= 1)
            def _():
                pltpu.make_async_copy(
                    hbuf.at[nb], acc.at[ibuf.at[ns, 1]], sem_s.at[nb]).wait()
            issue_body(j + 2)

            @pl.when(j + 3 < NBLK)
            def _():
                issue_idx(j + 3)
        return 0

    lax.fori_loop(0, NBLK, block, 0)
    for t in range(3):
        pltpu.make_async_copy(hbuf.at[t], acc.at[ibuf.at[t, 1]],
                              sem_s.at[t]).wait()
    plsc.subcore_barrier()

    def och(c, _):
        r0 = sid * RPT + c * BE
        pltpu.sync_copy(acc.at[pl.ds(r0, BE)], hbuf.at[0])
        pltpu.sync_copy(hbuf.at[0], out_hbm.at[cid, pl.ds(r0, BE)])
        return 0

    lax.fori_loop(0, nch, och, 0)


BEP = 2000


def _eproj_body(ea_ref, w_ref, b_ref, o_ref):
    o_ref[...] = (jnp.dot(ea_ref[...], w_ref[...],
                          preferred_element_type=jnp.float32) + b_ref[...])


def _eproj(ea, w, b):
    return pl.pallas_call(
        _eproj_body,
        grid=(E // BEP,),
        in_specs=[pl.BlockSpec((BEP, DE), lambda i: (i, 0)),
                  pl.BlockSpec((DE, D), lambda i: (0, 0)),
                  pl.BlockSpec((1, D), lambda i: (0, 0))],
        out_specs=pl.BlockSpec((BEP, D), lambda i: (i, 0)),
        out_shape=jax.ShapeDtypeStruct((E, D), jnp.float32),
    )(ea, w, b.reshape(1, D))


def _mlp_body(aggr_ref, h_ref, w1_ref, b1_ref, w2_ref, b2_ref,
              g_ref, bt_ref, o_ref):
    z = aggr_ref[0] + aggr_ref[1] + h_ref[...]
    a = jnp.maximum(jnp.dot(z, w1_ref[...],
                            preferred_element_type=jnp.float32) + b1_ref[...],
                    0.0)
    zz = jnp.dot(a, w2_ref[...],
                 preferred_element_type=jnp.float32) + b2_ref[...]
    mu = jnp.mean(zz, axis=0, keepdims=True)
    c = zz - mu
    var = jnp.mean(c * c, axis=0, keepdims=True)
    o_ref[...] = jnp.maximum(
        c / jnp.sqrt(var + 1e-5) * g_ref[...] + bt_ref[...], 0.0)


def _mlp(aggr, h, p):
    return pl.pallas_call(
        _mlp_body,
        out_shape=jax.ShapeDtypeStruct((N, D), jnp.float32),
    )(aggr, h, p['w1'], p['b1'].reshape(1, 2 * D), p['w2'],
      p['b2'].reshape(1, D), p['gamma'].reshape(1, D), p['beta'].reshape(1, D))


def kernel(atom_type, chirality, edge_index, edge_attr, atom_emb, chir_emb,
           params):
    key = (atom_type.astype(jnp.int32) * NCHIR + chirality.astype(jnp.int32))
    combo = (atom_emb[:, None, :] + chir_emb[None, :, :]).reshape(-1, D)
    x = _embed_sc_build()(combo, key)
    ei = (edge_index.astype(jnp.int32)
          .reshape(2, NW, NBLK, BE).transpose(1, 2, 0, 3))
    h = x
    for p in params:
        e = _eproj(edge_attr, p['edge_w'], p['edge_b'])
        parts = _msg_aggr_sc_build()(h, e, ei)
        h = _mlp(parts, h, p)
    return h


# speedup vs baseline: 1.0418x; 1.0418x over previous
import functools

import jax
import jax.numpy as jnp
from jax import lax
from jax.experimental import pallas as pl
from jax.experimental.pallas import tpu as pltpu
from jax.experimental.pallas import tpu_sc as plsc

N = 10000
E = 320000
D = 128
DE = 16
NCHIR = 4

NC = 2
NS = 16
NW = NC * NS
EPW = E // NW
BE = 80
NBLK = EPW // BE
RPT = 640
NPW = 312

@functools.cache
def _embed_sc_build():
    mesh = plsc.VectorSubcoreMesh(core_axis_name="c", subcore_axis_name="s")
    return functools.partial(
        pl.kernel,
        mesh=mesh,
        out_type=jax.ShapeDtypeStruct((N, D), jnp.float32),
        scratch_types=[
            pltpu.VMEM((NPW,), jnp.int32),
            pltpu.VMEM((NPW, D), jnp.float32),
            pltpu.VMEM((16,), jnp.int32),
            pltpu.VMEM((16, D), jnp.float32),
            pltpu.SemaphoreType.DMA,
        ],
    )(_embed_sc_body)


def _embed_sc_body(tbl_hbm, key_hbm, x_hbm, idx_v, rows_v, idx2_v, rows2_v,
                   sem):
    wid = lax.axis_index("s") * NC + lax.axis_index("c")
    base = wid * NPW
    pltpu.sync_copy(key_hbm.at[pl.ds(base, NPW)], idx_v)
    for c in range(3):
        pltpu.async_copy(tbl_hbm.at[idx_v.at[pl.ds(c * 104, 104)]],
                         rows_v.at[pl.ds(c * 104, 104)], sem).wait()
    pltpu.sync_copy(rows_v, x_hbm.at[pl.ds(base, NPW)])

    @pl.when(wid == NW - 1)
    def _():
        pltpu.sync_copy(key_hbm.at[pl.ds(NW * NPW, 16)], idx2_v)
        pltpu.async_copy(tbl_hbm.at[idx2_v], rows2_v, sem).wait()
        pltpu.sync_copy(rows2_v, x_hbm.at[pl.ds(NW * NPW, 16)])


@functools.cache
def _msg_aggr_sc_build():
    mesh = plsc.VectorSubcoreMesh(core_axis_name="c", subcore_axis_name="s")
    return functools.partial(
        pl.kernel,
        mesh=mesh,
        out_type=jax.ShapeDtypeStruct((NC, N, D), jnp.float32),
        scratch_types=[
            pltpu.VMEM((4, 2, BE), jnp.int32),
            pltpu.VMEM((2, BE, D), jnp.float32),
            pltpu.VMEM((2, BE, D), jnp.float32),
            pltpu.VMEM_SHARED((N, D), jnp.float32),
            pltpu.SemaphoreType.DMA((4,)),
            pltpu.SemaphoreType.DMA((2,)),
            pltpu.SemaphoreType.DMA((2,)),
            pltpu.SemaphoreType.DMA((2,)),
        ],
    )(_msg_aggr_sc_body)


def _msg_aggr_sc_body(h_hbm, e_hbm, ei_hbm, out_hbm,
                      ibuf, hbuf, ebuf, acc, sem_i, sem_g, sem_e, sem_s):
    cid = lax.axis_index("c")
    sid = lax.axis_index("s")
    wid = sid * NC + cid
    nrows = jnp.where(sid == NS - 1, 400, RPT)
    nch = nrows // BE
    zv = jnp.zeros((16,), jnp.float32)

    def zrow(r, _):
        for g in range(8):
            hbuf[0, r, pl.ds(g * 16, 16)] = zv
        return 0

    lax.fori_loop(0, BE, zrow, 0)

    def zch(c, _):
        pltpu.sync_copy(hbuf.at[0], acc.at[pl.ds(sid * RPT + c * BE, BE)])
        return 0

    lax.fori_loop(0, nch, zch, 0)

    def issue_idx(j):
        pltpu.async_copy(ei_hbm.at[wid, j], ibuf.at[j % 4], sem_i.at[j % 4])

    def issue_body(j):
        s = j % 4
        b = j % 2
        pltpu.async_copy(h_hbm.at[ibuf.at[s, 0]], hbuf.at[b], sem_g.at[b])
        pltpu.async_copy(e_hbm.at[pl.ds(wid * EPW + j * BE, BE)],
                         ebuf.at[b], sem_e.at[b])

    issue_idx(0)
    pltpu.make_async_copy(ei_hbm.at[wid, 0], ibuf.at[0], sem_i.at[0]).wait()
    issue_body(0)
    issue_idx(1)
    plsc.subcore_barrier()

    def block(j, _):
        s = j % 4
        ns = (j + 1) % 4
        b = j % 2
        nb = (j + 1) % 2
        pltpu.make_async_copy(h_hbm.at[ibuf.at[s, 0]], hbuf.at[b],
                              sem_g.at[b]).wait()
        pltpu.make_async_copy(e_hbm.at[pl.ds(wid * EPW + j * BE, BE)],
                              ebuf.at[b], sem_e.at[b]).wait()

        @pl.when(j + 1 < NBLK)
        def _():
            pltpu.make_async_copy(ei_hbm.at[wid, j + 1], ibuf.at[ns],
                                  sem_i.at[ns]).wait()
            @pl.when(j >= 1)
            def _():
                pltpu.make_async_copy(
                    hbuf.at[nb], acc.at[ibuf.at[ns, 1]], sem_s.at[nb]).wait()
            issue_body(j + 1)

            @pl.when(j + 2 < NBLK)
            def _():
                issue_idx(j + 2)

        for bb in range(2):
            @pl.when(b == bb)
            def _(bb=bb):
                def mrow(r, _):
                    sls = [pl.ds(g * 16, 16) for g in range(8)]
                    hv = [hbuf[bb, r, sl] for sl in sls]
                    ev = [ebuf[bb, r, sl] for sl in sls]
                    mv = [jnp.maximum(h + e, 0.0) for h, e in zip(hv, ev)]
                    for sl, m in zip(sls, mv):
                        hbuf[bb, r, sl] = m
                    return 0

                lax.fori_loop(0, BE, mrow, 0)
        pltpu.async_copy(hbuf.at[b], acc.at[ibuf.at[s, 1]], sem_s.at[b],
                         add=True)
        return 0

    lax.fori_loop(0, NBLK, block, 0)
    pltpu.make_async_copy(hbuf.at[0], acc.at[ibuf.at[0, 1]],
                          sem_s.at[0]).wait()
    pltpu.make_async_copy(hbuf.at[1], acc.at[ibuf.at[1, 1]],
                          sem_s.at[1]).wait()
    plsc.subcore_barrier()

    def och(c, _):
        r0 = sid * RPT + c * BE
        pltpu.sync_copy(acc.at[pl.ds(r0, BE)], hbuf.at[0])
        pltpu.sync_copy(hbuf.at[0], out_hbm.at[cid, pl.ds(r0, BE)])
        return 0

    lax.fori_loop(0, nch, och, 0)


BEP = 4000


def _eproj_body(ea_ref, w_ref, b_ref, o_ref):
    o_ref[...] = (jnp.dot(ea_ref[...], w_ref[...],
                          preferred_element_type=jnp.float32) + b_ref[...])


def _eproj(ea, w, b):
    return pl.pallas_call(
        _eproj_body,
        grid=(E // BEP,),
        in_specs=[pl.BlockSpec((BEP, DE), lambda i: (i, 0)),
                  pl.BlockSpec((DE, D), lambda i: (0, 0)),
                  pl.BlockSpec((1, D), lambda i: (0, 0))],
        out_specs=pl.BlockSpec((BEP, D), lambda i: (i, 0)),
        out_shape=jax.ShapeDtypeStruct((E, D), jnp.float32),
    )(ea, w, b.reshape(1, D))


def _mlp_body(aggr_ref, h_ref, w1_ref, b1_ref, w2_ref, b2_ref,
              g_ref, bt_ref, o_ref):
    z = aggr_ref[0] + aggr_ref[1] + h_ref[...]
    a = jnp.maximum(jnp.dot(z, w1_ref[...],
                            preferred_element_type=jnp.float32) + b1_ref[...],
                    0.0)
    zz = jnp.dot(a, w2_ref[...],
                 preferred_element_type=jnp.float32) + b2_ref[...]
    mu = jnp.mean(zz, axis=0, keepdims=True)
    c = zz - mu
    var = jnp.mean(c * c, axis=0, keepdims=True)
    o_ref[...] = jnp.maximum(
        c / jnp.sqrt(var + 1e-5) * g_ref[...] + bt_ref[...], 0.0)


def _mlp(aggr, h, p):
    return pl.pallas_call(
        _mlp_body,
        out_shape=jax.ShapeDtypeStruct((N, D), jnp.float32),
    )(aggr, h, p['w1'], p['b1'].reshape(1, 2 * D), p['w2'],
      p['b2'].reshape(1, D), p['gamma'].reshape(1, D), p['beta'].reshape(1, D))


def kernel(atom_type, chirality, edge_index, edge_attr, atom_emb, chir_emb,
           params):
    key = (atom_type.astype(jnp.int32) * NCHIR + chirality.astype(jnp.int32))
    combo = (atom_emb[:, None, :] + chir_emb[None, :, :]).reshape(-1, D)
    x = _embed_sc_build()(combo, key)
    ei = (edge_index.astype(jnp.int32)
          .reshape(2, NW, NBLK, BE).transpose(1, 2, 0, 3))
    h = x
    for p in params:
        e = _eproj(edge_attr, p['edge_w'], p['edge_b'])
        parts = _msg_aggr_sc_build()(h, e, ei)
        h = _mlp(parts, h, p)
    return h
